# speedup vs baseline: 154.0883x; 154.0883x over previous
import functools

import jax
import jax.numpy as jnp
from jax import lax
from jax.experimental import pallas as pl
from jax.experimental.pallas import tpu as pltpu
from jax.experimental.pallas import tpu_sc as plsc

N_DOF = 261121
NNZ = 1827847
NW = 32
C = 4096
N_PAD = 262144
U_SL = N_PAD // 16

CHUNKS_PER_W = -(-NNZ // (NW * C))
NNZ_PAD = NW * C * CHUNKS_PER_W
PER_W = NNZ_PAD // NW

_mesh = plsc.VectorSubcoreMesh(core_axis_name="c", subcore_axis_name="s")


@functools.partial(
    pl.kernel,
    out_type=jax.ShapeDtypeStruct((2, N_PAD), jnp.float32),
    mesh=_mesh,
    scratch_types=[
        pltpu.VMEM_SHARED((N_PAD,), jnp.float32),
        pltpu.VMEM_SHARED((N_PAD,), jnp.float32),
        pltpu.VMEM((C,), jnp.int32),
        pltpu.VMEM((C,), jnp.int32),
        pltpu.VMEM((C,), jnp.float32),
        pltpu.VMEM((C,), jnp.float32),
        pltpu.VMEM((C,), jnp.float32),
        pltpu.SemaphoreType.DMA,
    ],
)
def _spmv_sc(u_hbm, rows_hbm, cols_hbm, vals_hbm, out_hbm,
             u_s, acc_s, rows_v, cols_v, vals_v, g_v, z_v, sem):
    cid = lax.axis_index("c")
    sid = lax.axis_index("s")

    pltpu.sync_copy(u_hbm.at[pl.ds(sid * U_SL, U_SL)],
                    u_s.at[pl.ds(sid * U_SL, U_SL)])

    def zset(i, _):
        z_v[pl.ds(i * 16, 16)] = jnp.zeros((16,), jnp.float32)
        return 0
    lax.fori_loop(0, C // 16, zset, 0)

    def zcpy(j, _):
        pltpu.sync_copy(z_v, acc_s.at[pl.ds(sid * U_SL + j * C, C)])
        return 0
    lax.fori_loop(0, U_SL // C, zcpy, 0)

    plsc.subcore_barrier()

    wid = sid * 2 + cid
    base = wid * PER_W

    def chunk(t, _):
        off = base + t * C
        pltpu.sync_copy(rows_hbm.at[pl.ds(off, C)], rows_v)
        pltpu.sync_copy(cols_hbm.at[pl.ds(off, C)], cols_v)
        pltpu.sync_copy(vals_hbm.at[pl.ds(off, C)], vals_v)
        pltpu.async_copy(u_s.at[cols_v], g_v, sem).wait()

        def mul(i, _):
            s = pl.ds(i * 16, 16)
            g_v[s] = g_v[s] * vals_v[s]
            return 0
        lax.fori_loop(0, C // 16, mul, 0)

        pltpu.sync_copy(g_v, acc_s.at[rows_v], add=True)
        return 0
    lax.fori_loop(0, CHUNKS_PER_W, chunk, 0)

    plsc.subcore_barrier()
    pltpu.sync_copy(acc_s.at[pl.ds(sid * U_SL, U_SL)],
                    out_hbm.at[cid, pl.ds(sid * U_SL, U_SL)])


def _add_body(p_ref, o_ref):
    o_ref[...] = p_ref[0] + p_ref[1]


def kernel(u, A_rows, A_cols, A_vals):
    u_p = jnp.concatenate(
        [u[:, 0], jnp.zeros((N_PAD - N_DOF,), jnp.float32)])
    pad = NNZ_PAD - NNZ
    pad_idx = (jnp.arange(pad, dtype=jnp.int32) * 37) % N_DOF
    rows = jnp.concatenate([A_rows.astype(jnp.int32), pad_idx])
    cols = jnp.concatenate([A_cols.astype(jnp.int32), pad_idx])
    vals = jnp.concatenate([A_vals, jnp.zeros((pad,), jnp.float32)])

    partials = _spmv_sc(u_p, rows, cols, vals)

    summed = pl.pallas_call(
        _add_body,
        out_shape=jax.ShapeDtypeStruct((2048, 128), jnp.float32),
    )(partials.reshape(2, 2048, 128))
    return summed.reshape(N_PAD)[:N_DOF, None]


# speedup vs baseline: 187.7850x; 1.2187x over previous
import functools

import jax
import jax.numpy as jnp
from jax import lax
from jax.experimental import pallas as pl
from jax.experimental.pallas import tpu as pltpu
from jax.experimental.pallas import tpu_sc as plsc

N_DOF = 261121
NNZ = 1827847
NW = 32
C = 8192
N_PAD = 262144
U_SL = N_PAD // 16

FULL_CHUNKS = NNZ // C
TAIL = NNZ - FULL_CHUNKS * C
CPW = -(-FULL_CHUNKS // NW)

_mesh = plsc.VectorSubcoreMesh(core_axis_name="c", subcore_axis_name="s")


@functools.partial(
    pl.kernel,
    out_type=jax.ShapeDtypeStruct((2, N_PAD), jnp.float32),
    mesh=_mesh,
    scratch_types=[
        pltpu.VMEM_SHARED((N_PAD,), jnp.float32),
        pltpu.VMEM_SHARED((N_PAD,), jnp.float32),
        pltpu.VMEM((C,), jnp.int32),
        pltpu.VMEM((C,), jnp.int32),
        pltpu.VMEM((C,), jnp.float32),
        pltpu.VMEM((C,), jnp.float32),
        pltpu.VMEM((C,), jnp.float32),
        pltpu.SemaphoreType.DMA,
    ],
)
def _spmv_sc(u_hbm, rows_hbm, cols_hbm, vals_hbm,
             trows_hbm, tcols_hbm, tvals_hbm, out_hbm,
             u_s, acc_s, rows_v, cols_v, vals_v, g_v, z_v, sem):
    cid = lax.axis_index("c")
    sid = lax.axis_index("s")

    pltpu.sync_copy(u_hbm.at[pl.ds(sid * U_SL, U_SL)],
                    u_s.at[pl.ds(sid * U_SL, U_SL)])

    def zset(i, _):
        z_v[pl.ds(i * 16, 16)] = jnp.zeros((16,), jnp.float32)
        return 0
    lax.fori_loop(0, C // 16, zset, 0)

    def zcpy(j, _):
        pltpu.sync_copy(z_v, acc_s.at[pl.ds(sid * U_SL + j * C, C)])
        return 0
    lax.fori_loop(0, U_SL // C, zcpy, 0)

    plsc.subcore_barrier()

    wid = sid * 2 + cid

    def do_chunk(rows_src, cols_src, vals_src, off):
        pltpu.sync_copy(rows_src.at[pl.ds(off, C)], rows_v)
        pltpu.sync_copy(cols_src.at[pl.ds(off, C)], cols_v)
        pltpu.sync_copy(vals_src.at[pl.ds(off, C)], vals_v)
        pltpu.async_copy(u_s.at[cols_v], g_v, sem).wait()

        def mul(i, _):
            s = pl.ds(i * 16, 16)
            g_v[s] = g_v[s] * vals_v[s]
            return 0
        lax.fori_loop(0, C // 16, mul, 0)

        pltpu.sync_copy(g_v, acc_s.at[rows_v], add=True)

    base = wid * CPW * C
    nc = jnp.where(wid < NW - 1, CPW, FULL_CHUNKS - (NW - 1) * CPW)

    def chunk(t, _):
        do_chunk(rows_hbm, cols_hbm, vals_hbm, base + t * C)
        return 0
    lax.fori_loop(0, nc, chunk, 0)

    @pl.when(wid == NW - 1)
    def _tail():
        do_chunk(trows_hbm, tcols_hbm, tvals_hbm, 0)

    plsc.subcore_barrier()
    pltpu.sync_copy(acc_s.at[pl.ds(sid * U_SL, U_SL)],
                    out_hbm.at[cid, pl.ds(sid * U_SL, U_SL)])


def _add_body(p_ref, o_ref):
    o_ref[...] = p_ref[0] + p_ref[1]


def kernel(u, A_rows, A_cols, A_vals):
    u_p = jnp.concatenate(
        [u[:, 0], jnp.zeros((N_PAD - N_DOF,), jnp.float32)])

    pad = C - TAIL
    pad_idx = (jnp.arange(pad, dtype=jnp.int32) * 37) % N_DOF
    trows = jnp.concatenate(
        [lax.dynamic_slice(A_rows.astype(jnp.int32), (FULL_CHUNKS * C,), (TAIL,)),
         pad_idx])
    tcols = jnp.concatenate(
        [lax.dynamic_slice(A_cols.astype(jnp.int32), (FULL_CHUNKS * C,), (TAIL,)),
         pad_idx])
    tvals = jnp.concatenate(
        [lax.dynamic_slice(A_vals, (FULL_CHUNKS * C,), (TAIL,)),
         jnp.zeros((pad,), jnp.float32)])

    partials = _spmv_sc(u_p, A_rows.astype(jnp.int32), A_cols.astype(jnp.int32),
                        A_vals, trows, tcols, tvals)

    summed = pl.pallas_call(
        _add_body,
        out_shape=jax.ShapeDtypeStruct((2048, 128), jnp.float32),
    )(partials.reshape(2, 2048, 128))
    return summed.reshape(N_PAD)[:N_DOF, None]
